# baseline (speedup 1.0000x reference)
import jax
import jax.numpy as jnp
from jax.experimental import pallas as pl

_BLK = 25000


def _mlp_body(x_ref, w_ref, w2_ref, o_ref):
    i = pl.program_id(0)
    k = x_ref.shape[1]
    w1 = w_ref[:k, :]
    b1 = w_ref[k : k + 1, :]
    b2 = w_ref[k + 1 : k + 2, 0:1]
    h = jnp.dot(x_ref[...], w1, preferred_element_type=jnp.float32)
    h = jnp.maximum(h + b1, 0.0)
    y = jnp.dot(h, w2_ref[...], preferred_element_type=jnp.float32)
    row = jnp.transpose(y, (1, 0)) + b2
    o_ref[0, pl.ds(i, 1), :] = row


def kernel(X, W1, b1, W2, b2):
    n, k = X.shape
    d = W1.shape[1]
    blk = _BLK if n % _BLK == 0 else 8
    pad = (-n) % blk
    if pad:
        X = jnp.pad(X, ((0, pad), (0, 0)))
    npad = n + pad
    nsteps = npad // blk

    wpack = jnp.concatenate(
        [
            W1,
            b1.reshape(1, d),
            jnp.broadcast_to(b2.reshape(1, 1), (1, d)),
        ],
        axis=0,
    )

    out = pl.pallas_call(
        _mlp_body,
        grid=(nsteps,),
        in_specs=[
            pl.BlockSpec((blk, k), lambda i: (i, 0)),
            pl.BlockSpec((k + 2, d), lambda i: (0, 0)),
            pl.BlockSpec((d, 1), lambda i: (0, 0)),
        ],
        out_specs=pl.BlockSpec((1, nsteps, blk), lambda i: (0, 0, 0)),
        out_shape=jax.ShapeDtypeStruct((1, nsteps, blk), jnp.float32),
    )(X, wpack, W2)
    out = out.reshape(npad, 1)
    return out[:n] if pad else out


# speedup vs baseline: 1.4059x; 1.4059x over previous
import jax
import jax.numpy as jnp
from jax.experimental import pallas as pl

_BLK = 20000


def _mlp_body(x_ref, w_ref, w2_ref, o_ref):
    i = pl.program_id(0)
    k = x_ref.shape[1]
    w1 = w_ref[:k, :]
    b1 = w_ref[k : k + 1, :]
    b2 = w_ref[k + 1 : k + 2, 0:1]
    h = jnp.dot(x_ref[...], w1, preferred_element_type=jnp.float32)
    h = jnp.maximum(h + b1, 0.0)
    y = jnp.dot(h, w2_ref[...], preferred_element_type=jnp.float32)
    row = jnp.transpose(y, (1, 0)) + b2
    o_ref[0, pl.ds(i, 1), :] = row


def kernel(X, W1, b1, W2, b2):
    n, k = X.shape
    d = W1.shape[1]
    blk = _BLK if n % _BLK == 0 else 8
    pad = (-n) % blk
    if pad:
        X = jnp.pad(X, ((0, pad), (0, 0)))
    npad = n + pad
    nsteps = npad // blk

    wpack = jnp.concatenate(
        [
            W1,
            b1.reshape(1, d),
            jnp.broadcast_to(b2.reshape(1, 1), (1, d)),
        ],
        axis=0,
    )

    out = pl.pallas_call(
        _mlp_body,
        grid=(nsteps,),
        in_specs=[
            pl.BlockSpec((blk, k), lambda i: (i, 0)),
            pl.BlockSpec((k + 2, d), lambda i: (0, 0)),
            pl.BlockSpec((d, 1), lambda i: (0, 0)),
        ],
        out_specs=pl.BlockSpec((1, nsteps, blk), lambda i: (0, 0, 0)),
        out_shape=jax.ShapeDtypeStruct((1, nsteps, blk), jnp.float32),
    )(X, wpack, W2)
    out = out.reshape(npad, 1)
    return out[:n] if pad else out
